# Initial kernel scaffold; baseline (speedup 1.0000x reference)
import jax
import jax.numpy as jnp
from jax.experimental import pallas as pl


def kernel(p, cloud_pos, npc_geo_feats, B_embed, ptsW0, ptsb0, ptsW1, ptsb1, ptsW2, ptsb2, ptsW3, ptsb3, ptsW4, ptsb4, fcW0, fcb0, fcW1, fcb1, fcW2, fcb2, fcW3, fcb3, fcW4, fcb4, outW, outb):
    raise NotImplementedError("write your pallas kernel here")


# speedup vs baseline: 1.0823x; 1.0823x over previous
import functools
import math

import jax
import jax.numpy as jnp
from jax import lax
from jax.experimental import pallas as pl
from jax.experimental.pallas import tpu as pltpu
from jax.experimental.pallas import tpu_sc as plsc

N_PTS = 4096
M_CLOUD = 20000
C_DIM = 32
HIDDEN = 128
EMB = 93
NN = 8
R2 = 1.0

MP = 20480
WCH = 512
NCH = MP // WCH
TQ = 256
TB = 512
KPAD = 8

_INF = 3.0e38
_IBIG = 2**30

_SC_NW = 32
_G_CH = 8
_G_W = 128


def _topk_body(p_ref, cpos_ref, bd_ref, bi_ref):
    p = p_ref[...]
    pn = jnp.sum(p * p, axis=1, keepdims=True)

    def chunk(ch, carry):
        bd, bi = carry
        cpos = cpos_ref[ch]
        cn = jnp.sum(cpos * cpos, axis=0, keepdims=True)
        d = pn + cn - 2.0 * jnp.dot(p, cpos, preferred_element_type=jnp.float32)
        ii = ch * WCH + lax.broadcasted_iota(jnp.int32, (TQ, WCH), 1)
        nd, ni = [], []
        for _ in range(NN):
            m = jnp.minimum(jnp.min(d, axis=1, keepdims=True),
                            jnp.min(bd, axis=1, keepdims=True))
            seld = d == m
            selb = bd == m
            im = jnp.minimum(
                jnp.min(jnp.where(seld, ii, _IBIG), axis=1, keepdims=True),
                jnp.min(jnp.where(selb, bi, _IBIG), axis=1, keepdims=True))
            nd.append(m)
            ni.append(im)
            d = jnp.where(seld & (ii == im), _INF, d)
            bd = jnp.where(selb & (bi == im), _INF, bd)
        return jnp.concatenate(nd, axis=1), jnp.concatenate(ni, axis=1)

    bd0 = jnp.full((TQ, NN), _INF, jnp.float32)
    bi0 = jnp.full((TQ, NN), _IBIG, jnp.int32)
    bd, bi = lax.fori_loop(0, NCH, chunk, (bd0, bi0))
    bd_ref[...] = jnp.maximum(bd, 0.0)
    bi_ref[...] = bi


def _softplus100(z):
    z = 100.0 * z
    return (jnp.maximum(z, 0.0) + jnp.log1p(jnp.exp(-jnp.abs(z)))) * 0.01


def _mlp_body(p_ref, B_ref, d_ref, g_ref, E_ref, S_ref, nz_ref,
              w0_ref, w1_ref, w2_ref, w3e_ref, w3h_ref, w4_ref,
              b0_ref, b1_ref, b2_ref, b3_ref, b4_ref,
              f0_ref, f1_ref, f2_ref, f3_ref, f4_ref,
              g0_ref, g1_ref, g2_ref, g3_ref, g4_ref,
              ow_ref, ob_ref, o_ref):
    f32 = jnp.float32
    p = p_ref[...]
    emb = jnp.sin((2.0 * math.pi) *
                  jnp.dot(p, B_ref[...], preferred_element_type=f32))
    D = d_ref[...]
    w = 1.0 / (D + 1e-10)
    w = jnp.where(D > R2, 0.0, w)
    wsum = jnp.maximum(jnp.sum(jnp.abs(w), axis=1, keepdims=True), 1e-12)
    w = w / wsum
    we = jnp.dot(w, E_ref[...], preferred_element_type=f32)
    c = jnp.dot(we * g_ref[...], S_ref[...], preferred_element_type=f32)
    cnt = jnp.sum(jnp.where(D <= R2, 1.0, 0.0), axis=1, keepdims=True)
    c = jnp.where(cnt >= 2.0, c, nz_ref[...])

    def blk(h, w_r, b_r, f_r, fb_r):
        z = jnp.dot(h, w_r[...], preferred_element_type=f32) + b_r[...]
        return (_softplus100(z) +
                jnp.dot(c, f_r[...], preferred_element_type=f32) + fb_r[...])

    h = blk(emb, w0_ref, b0_ref, f0_ref, g0_ref)
    h = blk(h, w1_ref, b1_ref, f1_ref, g1_ref)
    h = blk(h, w2_ref, b2_ref, f2_ref, g2_ref)
    z = (jnp.dot(emb, w3e_ref[...], preferred_element_type=f32) +
         jnp.dot(h, w3h_ref[...], preferred_element_type=f32) + b3_ref[...])
    h = (_softplus100(z) +
         jnp.dot(c, f3_ref[...], preferred_element_type=f32) + g3_ref[...])
    h = blk(h, w4_ref, b4_ref, f4_ref, g4_ref)
    o_ref[...] = jnp.dot(h, ow_ref[...], preferred_element_type=f32) + ob_ref[...]


def _make_sc_gather():
    mesh = plsc.VectorSubcoreMesh(core_axis_name="c", subcore_axis_name="s")

    @functools.partial(
        pl.kernel,
        mesh=mesh,
        out_type=jax.ShapeDtypeStruct((_SC_NW, _G_CH, _G_W, C_DIM), jnp.float32),
        scratch_types=[
            pltpu.VMEM((_G_CH, _G_W), jnp.int32),
            pltpu.VMEM((_G_CH, _G_W, C_DIM), jnp.float32),
            pltpu.SemaphoreType.DMA,
        ],
        compiler_params=pltpu.CompilerParams(use_tc_tiling_on_sc=False),
    )
    def gather(table_hbm, idx_hbm, out_hbm, idx_v, rows_v, sem):
        wid = lax.axis_index("s") * 2 + lax.axis_index("c")
        pltpu.sync_copy(idx_hbm.at[wid], idx_v)
        copies = [pltpu.async_copy(table_hbm.at[idx_v.at[j]], rows_v.at[j], sem)
                  for j in range(_G_CH)]
        for cp in copies:
            cp.wait()
        pltpu.sync_copy(rows_v, out_hbm.at[wid])

    return gather


@functools.cache
def _sc_gather_fn():
    return _make_sc_gather()


def _topk_call(p8, cpos_c):
    return pl.pallas_call(
        _topk_body,
        grid=(N_PTS // TQ,),
        in_specs=[
            pl.BlockSpec((TQ, KPAD), lambda i: (i, 0)),
            pl.BlockSpec((NCH, KPAD, WCH), lambda i: (0, 0, 0)),
        ],
        out_specs=[
            pl.BlockSpec((TQ, NN), lambda i: (i, 0)),
            pl.BlockSpec((TQ, NN), lambda i: (i, 0)),
        ],
        out_shape=[
            jax.ShapeDtypeStruct((N_PTS, NN), jnp.float32),
            jax.ShapeDtypeStruct((N_PTS, NN), jnp.int32),
        ],
    )(p8, cpos_c)


def _mlp_call(p8, Bp, bd, G, E, S, nz, Ws, bs, Fs, gs, ow, ob):
    full2 = lambda a: pl.BlockSpec(a.shape, lambda i: (0, 0))
    in_specs = ([pl.BlockSpec((TB, KPAD), lambda i: (i, 0)),
                 full2(Bp),
                 pl.BlockSpec((TB, NN), lambda i: (i, 0)),
                 pl.BlockSpec((TB, NN * C_DIM), lambda i: (i, 0)),
                 full2(E), full2(S), full2(nz)]
                + [full2(W) for W in Ws]
                + [full2(b) for b in bs]
                + [full2(F) for F in Fs]
                + [full2(g) for g in gs]
                + [full2(ow), full2(ob)])
    return pl.pallas_call(
        _mlp_body,
        grid=(N_PTS // TB,),
        in_specs=in_specs,
        out_specs=pl.BlockSpec((TB, 1), lambda i: (i, 0)),
        out_shape=jax.ShapeDtypeStruct((N_PTS, 1), jnp.float32),
    )(p8, Bp, bd, G, E, S, nz, *Ws, *bs, *Fs, *gs, ow, ob)


def kernel(p, cloud_pos, npc_geo_feats, B_embed,
           ptsW0, ptsb0, ptsW1, ptsb1, ptsW2, ptsb2, ptsW3, ptsb3,
           ptsW4, ptsb4, fcW0, fcb0, fcW1, fcb1, fcW2, fcb2, fcW3, fcb3,
           fcW4, fcb4, outW, outb):
    f32 = jnp.float32
    p8 = jnp.pad(p, ((0, 0), (0, KPAD - 3)))
    far = jnp.full((MP - M_CLOUD, 3), 1.0e4, f32)
    cp = jnp.concatenate([cloud_pos, far], axis=0)
    cp8 = jnp.pad(cp, ((0, 0), (0, KPAD - 3)))
    cpos_c = cp8.T.reshape(KPAD, NCH, WCH).transpose(1, 0, 2)

    bd, bi = _topk_call(p8, cpos_c)

    idx = bi.reshape(_SC_NW, _G_CH, _G_W)
    G = _sc_gather_fn()(npc_geo_feats, idx)
    G = G.reshape(N_PTS, NN * C_DIM)

    E = jnp.kron(jnp.eye(NN, dtype=f32), jnp.ones((1, C_DIM), f32))
    S = jnp.kron(jnp.ones((NN, 1), f32), jnp.eye(C_DIM, dtype=f32))
    nz = (0.01 * jax.random.normal(jax.random.key(42), (C_DIM,), f32)
          ).reshape(1, C_DIM)

    Bp = jnp.pad(B_embed, ((0, KPAD - 3), (0, HIDDEN - EMB)))
    W0p = jnp.pad(ptsW0, ((0, HIDDEN - EMB), (0, 0)))
    W3e = jnp.pad(ptsW3[:EMB], ((0, HIDDEN - EMB), (0, 0)))
    W3h = ptsW3[EMB:]
    Ws = [W0p, ptsW1, ptsW2, W3e, W3h, ptsW4]
    bs = [b.reshape(1, HIDDEN) for b in (ptsb0, ptsb1, ptsb2, ptsb3, ptsb4)]
    Fs = [fcW0, fcW1, fcW2, fcW3, fcW4]
    gs = [g.reshape(1, HIDDEN) for g in (fcb0, fcb1, fcb2, fcb3, fcb4)]
    ob = outb.reshape(1, 1)

    return _mlp_call(p8, Bp, bd, G, E, S, nz, Ws, bs, Fs, gs, outW, ob)
